# baseline (speedup 1.0000x reference)
import functools

import jax
import jax.numpy as jnp
from jax import lax
from jax.experimental import pallas as pl
from jax.experimental.pallas import tpu as pltpu
from jax.experimental.pallas import tpu_sc as plsc

D = 1024
NW = 32
CH = 16
RING = 4
EPS = 1e-5


def _make_gather(N):
    rows_per_w = N // NW
    nch = rows_per_w // CH
    assert nch % RING == 0
    mesh = plsc.VectorSubcoreMesh(core_axis_name="c", subcore_axis_name="s")

    @functools.partial(
        pl.kernel,
        mesh=mesh,
        compiler_params=pltpu.CompilerParams(needs_layout_passes=False),
        out_type=jax.ShapeDtypeStruct((N, D), jnp.float32),
        scratch_types=[
            pltpu.VMEM((rows_per_w,), jnp.int32),
            pltpu.VMEM((RING, CH, D), jnp.float32),
        ]
        + [pltpu.SemaphoreType.DMA] * (2 * RING),
    )
    def k(ids_hbm, tok_hbm, out_hbm, idx_v, buf, *sems):
        gsems = sems[:RING]
        osems = sems[RING:]
        wid = lax.axis_index("s") * 2 + lax.axis_index("c")
        base = wid * rows_per_w

        pltpu.sync_copy(ids_hbm.at[pl.ds(base, rows_per_w)], idx_v)

        def gather(c, slot):
            pltpu.async_copy(
                tok_hbm.at[idx_v.at[pl.ds(c * CH, CH)]], buf.at[slot],
                gsems[slot])

        for slot in range(RING):
            gather(slot, slot)

        def g_body(g, _):
            for slot in range(RING):
                c = RING * g + slot
                pltpu.make_async_copy(
                    tok_hbm.at[pl.ds(0, CH)], buf.at[slot],
                    gsems[slot]).wait()
                pltpu.async_copy(
                    buf.at[slot], out_hbm.at[pl.ds(base + c * CH, CH)],
                    osems[slot])

                @pl.when(c + RING < nch)
                def _():
                    pltpu.make_async_copy(
                        buf.at[slot], out_hbm.at[pl.ds(0, CH)],
                        osems[slot]).wait()
                    gather(c + RING, slot)
            return 0

        lax.fori_loop(0, nch // RING, g_body, 0)

        for slot in range(RING):
            pltpu.make_async_copy(
                buf.at[slot], out_hbm.at[pl.ds(0, CH)], osems[slot]).wait()

    return k


def _ln_body(x_ref, pos_ref, g_ref, b_ref, o_ref):
    x = x_ref[...] + pos_ref[...]
    mean = jnp.mean(x, axis=-1, keepdims=True)
    var = jnp.mean(x * x, axis=-1, keepdims=True) - mean * mean
    rstd = lax.rsqrt(var + EPS)
    o_ref[...] = (x - mean) * rstd * g_ref[...] + b_ref[...]


def _make_ln(B, S, rb):
    grid = (B, S // rb)
    return pl.pallas_call(
        _ln_body,
        grid=grid,
        in_specs=[
            pl.BlockSpec((rb, D), lambda b, s: (b * (S // rb) + s, 0)),
            pl.BlockSpec((rb, D), lambda b, s: (s, 0)),
            pl.BlockSpec((1, D), lambda b, s: (0, 0)),
            pl.BlockSpec((1, D), lambda b, s: (0, 0)),
        ],
        out_specs=pl.BlockSpec((rb, D), lambda b, s: (b * (S // rb) + s, 0)),
        out_shape=jax.ShapeDtypeStruct((B * S, D), jnp.float32),
        compiler_params=pltpu.CompilerParams(
            dimension_semantics=("parallel", "arbitrary")),
    )


def kernel(input_ids, token_table, pos_table, ln_gamma, ln_beta):
    B, S = input_ids.shape
    N = B * S
    ids = input_ids.reshape(N).astype(jnp.int32)
    gathered = _make_gather(N)(ids, token_table)
    out = _make_ln(B, S, 256)(
        gathered,
        pos_table[:S],
        ln_gamma.reshape(1, D),
        ln_beta.reshape(1, D),
    )
    return out.reshape(B, S, D)


# speedup vs baseline: 1.0118x; 1.0118x over previous
import functools

import jax
import jax.numpy as jnp
from jax import lax
from jax.experimental import pallas as pl
from jax.experimental.pallas import tpu as pltpu
from jax.experimental.pallas import tpu_sc as plsc

D = 1024
NW = 32
CH = 16
RING = 4
EPS = 1e-5


def _make_gather(N):
    rows_per_w = N // NW
    nch = rows_per_w // CH
    assert nch % RING == 0
    mesh = plsc.VectorSubcoreMesh(core_axis_name="c", subcore_axis_name="s")

    @functools.partial(
        pl.kernel,
        mesh=mesh,
        compiler_params=pltpu.CompilerParams(needs_layout_passes=False),
        out_type=jax.ShapeDtypeStruct((N, D), jnp.float32),
        scratch_types=[
            pltpu.VMEM((rows_per_w,), jnp.int32),
            pltpu.VMEM((RING, CH, D), jnp.float32),
        ]
        + [pltpu.SemaphoreType.DMA] * (2 * RING),
    )
    def k(ids_hbm, tok_hbm, out_hbm, idx_v, buf, *sems):
        gsems = sems[:RING]
        osems = sems[RING:]
        wid = lax.axis_index("s") * 2 + lax.axis_index("c")
        base = wid * rows_per_w

        pltpu.sync_copy(ids_hbm.at[pl.ds(base, rows_per_w)], idx_v)

        def gather(c, slot):
            pltpu.async_copy(
                tok_hbm.at[idx_v.at[pl.ds(c * CH, CH)]], buf.at[slot],
                gsems[slot])

        for slot in range(RING):
            gather(slot, slot)

        def g_body(g, _):
            for slot in range(RING):
                c = RING * g + slot
                pltpu.make_async_copy(
                    tok_hbm.at[pl.ds(0, CH)], buf.at[slot],
                    gsems[slot]).wait()
                pltpu.async_copy(
                    buf.at[slot], out_hbm.at[pl.ds(base + c * CH, CH)],
                    osems[slot])

                @pl.when(c + RING < nch)
                def _():
                    pltpu.make_async_copy(
                        buf.at[slot], out_hbm.at[pl.ds(0, CH)],
                        osems[slot]).wait()
                    gather(c + RING, slot)
            return 0

        lax.fori_loop(0, nch // RING, g_body, 0)

        for slot in range(RING):
            pltpu.make_async_copy(
                buf.at[slot], out_hbm.at[pl.ds(0, CH)], osems[slot]).wait()

    return k


def _ln_body(x_ref, pos_ref, g_ref, b_ref, o_ref):
    x = x_ref[...] + pos_ref[...]
    mean = jnp.mean(x, axis=-1, keepdims=True)
    var = jnp.mean(x * x, axis=-1, keepdims=True) - mean * mean
    rstd = lax.rsqrt(var + EPS)
    o_ref[...] = (x - mean) * rstd * g_ref[...] + b_ref[...]


def _make_ln(B, S, rb):
    grid = (S // rb, B)
    return pl.pallas_call(
        _ln_body,
        grid=grid,
        in_specs=[
            pl.BlockSpec((rb, D), lambda s, b: (b * (S // rb) + s, 0)),
            pl.BlockSpec((rb, D), lambda s, b: (s, 0)),
            pl.BlockSpec((1, D), lambda s, b: (0, 0)),
            pl.BlockSpec((1, D), lambda s, b: (0, 0)),
        ],
        out_specs=pl.BlockSpec((rb, D), lambda s, b: (b * (S // rb) + s, 0)),
        out_shape=jax.ShapeDtypeStruct((B * S, D), jnp.float32),
        compiler_params=pltpu.CompilerParams(
            dimension_semantics=("arbitrary", "arbitrary")),
    )


def kernel(input_ids, token_table, pos_table, ln_gamma, ln_beta):
    B, S = input_ids.shape
    N = B * S
    ids = input_ids.reshape(N).astype(jnp.int32)
    gathered = _make_gather(N)(ids, token_table)
    out = _make_ln(B, S, 256)(
        gathered,
        pos_table[:S],
        ln_gamma.reshape(1, D),
        ln_beta.reshape(1, D),
    )
    return out.reshape(B, S, D)
